# Initial kernel scaffold; baseline (speedup 1.0000x reference)
import jax
import jax.numpy as jnp
from jax.experimental import pallas as pl


def kernel(x, adjs, W_l1, b_l1, W_r1, W_l2, b_l2, W_r2):
    raise NotImplementedError("write your pallas kernel here")


# speedup vs baseline: 1.0314x; 1.0314x over previous
import functools

import jax
import jax.numpy as jnp
from jax.experimental import pallas as pl

N, F_IN, H, C = 10000, 128, 128, 64
BLK = 400


def _layer1_kernel(adj_ref, x_full_ref, x_blk_ref, wl1_ref, bl1_ref, wr1_ref,
                   wl2_ref, bl2_ref, wr2_ref, hw_ref, hr_ref):
    agg = jnp.dot(adj_ref[...], x_full_ref[...],
                  preferred_element_type=jnp.float32)
    out = jax.lax.dot_general(agg, wl1_ref[...],
                              (((1,), (1,)), ((), ())),
                              preferred_element_type=jnp.float32)
    out = out + bl1_ref[...]
    out = out + jax.lax.dot_general(x_blk_ref[...], wr1_ref[...],
                                    (((1,), (1,)), ((), ())),
                                    preferred_element_type=jnp.float32)
    denom = jnp.maximum(jnp.sum(jnp.abs(out), axis=1, keepdims=True), 1e-12)
    h = jnp.maximum(out / denom, 0.0)
    hw_ref[...] = jax.lax.dot_general(h, wl2_ref[...],
                                      (((1,), (1,)), ((), ())),
                                      preferred_element_type=jnp.float32)
    hr_ref[...] = jax.lax.dot_general(h, wr2_ref[...],
                                      (((1,), (1,)), ((), ())),
                                      preferred_element_type=jnp.float32) \
        + bl2_ref[...]


def _layer2_kernel(adj_ref, hw_full_ref, hr_blk_ref, out_ref):
    o = jnp.dot(adj_ref[...], hw_full_ref[...],
                preferred_element_type=jnp.float32)
    o = o + hr_blk_ref[...]
    m = jnp.max(o, axis=1, keepdims=True)
    lse = jnp.log(jnp.sum(jnp.exp(o - m), axis=1, keepdims=True))
    out_ref[...] = o - m - lse


@jax.jit
def kernel(x, adjs, W_l1, b_l1, W_r1, W_l2, b_l2, W_r2):
    nblk = N // BLK
    bl1 = b_l1.reshape(1, H)
    bl2 = b_l2.reshape(1, C)

    hw, hr = pl.pallas_call(
        _layer1_kernel,
        grid=(nblk,),
        in_specs=[
            pl.BlockSpec((BLK, N), lambda i: (i, 0)),
            pl.BlockSpec((N, F_IN), lambda i: (0, 0)),
            pl.BlockSpec((BLK, F_IN), lambda i: (i, 0)),
            pl.BlockSpec((H, F_IN), lambda i: (0, 0)),
            pl.BlockSpec((1, H), lambda i: (0, 0)),
            pl.BlockSpec((H, F_IN), lambda i: (0, 0)),
            pl.BlockSpec((C, H), lambda i: (0, 0)),
            pl.BlockSpec((1, C), lambda i: (0, 0)),
            pl.BlockSpec((C, H), lambda i: (0, 0)),
        ],
        out_specs=[
            pl.BlockSpec((BLK, C), lambda i: (i, 0)),
            pl.BlockSpec((BLK, C), lambda i: (i, 0)),
        ],
        out_shape=[
            jax.ShapeDtypeStruct((N, C), jnp.float32),
            jax.ShapeDtypeStruct((N, C), jnp.float32),
        ],
    )(adjs, x, x, W_l1, bl1, W_r1, W_l2, bl2, W_r2)

    out = pl.pallas_call(
        _layer2_kernel,
        grid=(nblk,),
        in_specs=[
            pl.BlockSpec((BLK, N), lambda i: (i, 0)),
            pl.BlockSpec((N, C), lambda i: (0, 0)),
            pl.BlockSpec((BLK, C), lambda i: (i, 0)),
        ],
        out_specs=pl.BlockSpec((BLK, C), lambda i: (i, 0)),
        out_shape=jax.ShapeDtypeStruct((N, C), jnp.float32),
    )(adjs, hw, hr)
    return out


# speedup vs baseline: 1.0799x; 1.0470x over previous
import jax
import jax.numpy as jnp
from jax.experimental import pallas as pl
from jax.experimental.pallas import tpu as pltpu

N, F_IN, H, C = 10000, 128, 128, 64
BLK = 400


def _sage_kernel(adj_ref, x_full_ref, x_blk_ref, wl1_ref, bl1_ref, wr1_ref,
                 wl2_ref, bl2_ref, wr2_ref, out_ref, hw_ref, hr_ref):
    p = pl.program_id(0)
    i = pl.program_id(1)

    @pl.when(p == 0)
    def _layer1():
        agg = jnp.dot(adj_ref[...], x_full_ref[...],
                      preferred_element_type=jnp.float32)
        o = jax.lax.dot_general(agg, wl1_ref[...],
                                (((1,), (1,)), ((), ())),
                                preferred_element_type=jnp.float32)
        o = o + bl1_ref[...]
        o = o + jax.lax.dot_general(x_blk_ref[...], wr1_ref[...],
                                    (((1,), (1,)), ((), ())),
                                    preferred_element_type=jnp.float32)
        denom = jnp.maximum(jnp.sum(jnp.abs(o), axis=1, keepdims=True), 1e-12)
        h = jnp.maximum(o / denom, 0.0)
        hw_ref[pl.ds(i * BLK, BLK), :] = jax.lax.dot_general(
            h, wl2_ref[...], (((1,), (1,)), ((), ())),
            preferred_element_type=jnp.float32)
        hr_ref[pl.ds(i * BLK, BLK), :] = jax.lax.dot_general(
            h, wr2_ref[...], (((1,), (1,)), ((), ())),
            preferred_element_type=jnp.float32) + bl2_ref[...]
        out_ref[...] = jnp.zeros((BLK, C), jnp.float32)

    @pl.when(p == 1)
    def _layer2():
        o = jnp.dot(adj_ref[...], hw_ref[...],
                    preferred_element_type=jnp.float32)
        o = o + hr_ref[pl.ds(i * BLK, BLK), :]
        m = jnp.max(o, axis=1, keepdims=True)
        lse = jnp.log(jnp.sum(jnp.exp(o - m), axis=1, keepdims=True))
        out_ref[...] = o - m - lse


@jax.jit
def kernel(x, adjs, W_l1, b_l1, W_r1, W_l2, b_l2, W_r2):
    nblk = N // BLK
    bl1 = b_l1.reshape(1, H)
    bl2 = b_l2.reshape(1, C)

    return pl.pallas_call(
        _sage_kernel,
        grid=(2, nblk),
        in_specs=[
            pl.BlockSpec((BLK, N), lambda p, i: (i, 0)),
            pl.BlockSpec((N, F_IN), lambda p, i: (0, 0)),
            pl.BlockSpec((BLK, F_IN), lambda p, i: (i, 0)),
            pl.BlockSpec((H, F_IN), lambda p, i: (0, 0)),
            pl.BlockSpec((1, H), lambda p, i: (0, 0)),
            pl.BlockSpec((H, F_IN), lambda p, i: (0, 0)),
            pl.BlockSpec((C, H), lambda p, i: (0, 0)),
            pl.BlockSpec((1, C), lambda p, i: (0, 0)),
            pl.BlockSpec((C, H), lambda p, i: (0, 0)),
        ],
        out_specs=pl.BlockSpec((BLK, C), lambda p, i: (i, 0)),
        out_shape=jax.ShapeDtypeStruct((N, C), jnp.float32),
        scratch_shapes=[
            pltpu.VMEM((N, C), jnp.float32),
            pltpu.VMEM((N, C), jnp.float32),
        ],
    )(adjs, x, x, W_l1, bl1, W_r1, W_l2, bl2, W_r2)


# speedup vs baseline: 1.0810x; 1.0010x over previous
import jax
import jax.numpy as jnp
from jax.experimental import pallas as pl
from jax.experimental.pallas import tpu as pltpu

N, F_IN, H, C = 10000, 128, 128, 64
BLK = 400


def _sage_kernel(adj_ref, x_full_ref, x_blk_ref, wl1_ref, bl1_ref, wr1_ref,
                 wl2_ref, bl2_ref, wr2_ref, out_ref, hw_ref, hr_ref):
    p = pl.program_id(0)
    i = pl.program_id(1)

    @pl.when(p == 0)
    def _layer1():
        agg = jnp.dot(adj_ref[...].astype(jnp.bfloat16),
                      x_full_ref[...].astype(jnp.bfloat16),
                      preferred_element_type=jnp.float32)
        o = jax.lax.dot_general(agg, wl1_ref[...],
                                (((1,), (1,)), ((), ())),
                                preferred_element_type=jnp.float32)
        o = o + bl1_ref[...]
        o = o + jax.lax.dot_general(x_blk_ref[...], wr1_ref[...],
                                    (((1,), (1,)), ((), ())),
                                    preferred_element_type=jnp.float32)
        denom = jnp.maximum(jnp.sum(jnp.abs(o), axis=1, keepdims=True), 1e-12)
        h = jnp.maximum(o / denom, 0.0)
        hw_ref[pl.ds(i * BLK, BLK), :] = jax.lax.dot_general(
            h, wl2_ref[...], (((1,), (1,)), ((), ())),
            preferred_element_type=jnp.float32).astype(jnp.bfloat16)
        hr_ref[pl.ds(i * BLK, BLK), :] = jax.lax.dot_general(
            h, wr2_ref[...], (((1,), (1,)), ((), ())),
            preferred_element_type=jnp.float32) + bl2_ref[...]
        out_ref[...] = jnp.zeros((BLK, C), jnp.float32)

    @pl.when(p == 1)
    def _layer2():
        o = jnp.dot(adj_ref[...].astype(jnp.bfloat16), hw_ref[...],
                    preferred_element_type=jnp.float32)
        o = o + hr_ref[pl.ds(i * BLK, BLK), :]
        m = jnp.max(o, axis=1, keepdims=True)
        lse = jnp.log(jnp.sum(jnp.exp(o - m), axis=1, keepdims=True))
        out_ref[...] = o - m - lse


@jax.jit
def kernel(x, adjs, W_l1, b_l1, W_r1, W_l2, b_l2, W_r2):
    nblk = N // BLK
    bl1 = b_l1.reshape(1, H)
    bl2 = b_l2.reshape(1, C)

    return pl.pallas_call(
        _sage_kernel,
        grid=(2, nblk),
        in_specs=[
            pl.BlockSpec((BLK, N), lambda p, i: (i, 0)),
            pl.BlockSpec((N, F_IN), lambda p, i: (0, 0)),
            pl.BlockSpec((BLK, F_IN), lambda p, i: (i, 0)),
            pl.BlockSpec((H, F_IN), lambda p, i: (0, 0)),
            pl.BlockSpec((1, H), lambda p, i: (0, 0)),
            pl.BlockSpec((H, F_IN), lambda p, i: (0, 0)),
            pl.BlockSpec((C, H), lambda p, i: (0, 0)),
            pl.BlockSpec((1, C), lambda p, i: (0, 0)),
            pl.BlockSpec((C, H), lambda p, i: (0, 0)),
        ],
        out_specs=pl.BlockSpec((BLK, C), lambda p, i: (i, 0)),
        out_shape=jax.ShapeDtypeStruct((N, C), jnp.float32),
        scratch_shapes=[
            pltpu.VMEM((N, C), jnp.bfloat16),
            pltpu.VMEM((N, C), jnp.float32),
        ],
    )(adjs, x, x, W_l1, bl1, W_r1, W_l2, bl2, W_r2)
